# initial kernel scaffold (unmeasured)
import jax
import jax.numpy as jnp
from jax import lax
from jax.experimental import pallas as pl
from jax.experimental.pallas import tpu as pltpu

N_DEV = 4
SQ = 256
D_MODEL = 1024
H = 8
DH = 128
SKV = 4096
N_RES = 4
KV_PER_RES = SKV // N_RES
SCALE = 0.08838834764831843


def _body(x_ref, wq_ref, k_ref, v_ref, wo_ref, out_ref,
          xg_ref, xres_ref, qres_ref, ctx_ref, part_ref, rs_ref,
          ag_send, ag_recv, rs_send, rs_recv):
    my = lax.axis_index("i")
    right = lax.rem(my + 1, N_DEV)
    left = lax.rem(my + 3, N_DEV)

    barrier = pltpu.get_barrier_semaphore()
    pl.semaphore_signal(barrier, inc=1, device_id=(left,),
                        device_id_type=pl.DeviceIdType.MESH)
    pl.semaphore_signal(barrier, inc=1, device_id=(right,),
                        device_id_type=pl.DeviceIdType.MESH)
    pl.semaphore_wait(barrier, 2)

    xg_ref[pl.ds(my * SQ, SQ), :] = x_ref[:, :]
    for h in range(N_DEV - 1):
        c = lax.rem(my - h + N_DEV, N_DEV)
        rdma = pltpu.make_async_remote_copy(
            src_ref=xg_ref.at[pl.ds(c * SQ, SQ), :],
            dst_ref=xg_ref.at[pl.ds(c * SQ, SQ), :],
            send_sem=ag_send.at[h],
            recv_sem=ag_recv.at[h],
            device_id=(right,),
            device_id_type=pl.DeviceIdType.MESH,
        )
        rdma.start()
        rdma.wait()

    for j in range(N_DEV):
        for r in range(N_RES):
            xres_ref[r * SQ + j * 64:r * SQ + j * 64 + 64, :] = (
                xg_ref[j * SQ + r * 64:j * SQ + r * 64 + 64, :])

    qres_ref[:, :] = jnp.dot(xres_ref[:, :], wq_ref[:, :],
                             preferred_element_type=jnp.float32)

    for r in range(N_RES):
        q_r = qres_ref[r * SQ:(r + 1) * SQ, :]
        for h in range(H):
            qh = q_r[:, h * DH:(h + 1) * DH]
            k_rh = k_ref[r, h]
            s = lax.dot_general(
                qh, k_rh, (((1,), (1,)), ((), ())),
                preferred_element_type=jnp.float32) * SCALE
            m = jnp.max(s, axis=1, keepdims=True)
            e = jnp.exp(s - m)
            p = e / jnp.sum(e, axis=1, keepdims=True)
            ctx_ref[:, h * DH:(h + 1) * DH] = jnp.dot(
                p, v_ref[r, h], preferred_element_type=jnp.float32)
        acc = jnp.dot(ctx_ref[:, :], wo_ref[:, :],
                      preferred_element_type=jnp.float32)
        for j in range(N_DEV):
            part_ref[j * SQ + r * 64:j * SQ + r * 64 + 64, :] = (
                acc[j * 64:(j + 1) * 64, :])

    for s in range(N_DEV - 1):
        cs = lax.rem(my + 3 - s + N_DEV, N_DEV)
        cr = lax.rem(my + 2 - s + N_DEV, N_DEV)
        rdma = pltpu.make_async_remote_copy(
            src_ref=part_ref.at[pl.ds(cs * SQ, SQ), :],
            dst_ref=rs_ref.at[pl.ds(s * SQ, SQ), :],
            send_sem=rs_send.at[s],
            recv_sem=rs_recv.at[s],
            device_id=(right,),
            device_id_type=pl.DeviceIdType.MESH,
        )
        rdma.start()
        rdma.wait()
        part_ref[pl.ds(cr * SQ, SQ), :] = (
            part_ref[pl.ds(cr * SQ, SQ), :] + rs_ref[s * SQ:(s + 1) * SQ, :])

    out_ref[:, :] = part_ref[pl.ds(my * SQ, SQ), :]


def kernel(x, Wq, K_ext, V_ext, Wo):
    my = lax.axis_index("i")

    x2 = x[0]
    K = K_ext[0]
    V = V_ext[0]
    Kh = lax.dynamic_slice_in_dim(K, my * H, H, axis=1)
    Vh = lax.dynamic_slice_in_dim(V, my * H, H, axis=1)
    Kp = Kh.reshape(16, N_RES, 64, H, DH).transpose(1, 3, 0, 2, 4).reshape(
        N_RES, H, KV_PER_RES, DH)
    Vp = Vh.reshape(16, N_RES, 64, H, DH).transpose(1, 3, 0, 2, 4).reshape(
        N_RES, H, KV_PER_RES, DH)

    out = pl.pallas_call(
        _body,
        out_shape=jax.ShapeDtypeStruct((SQ, D_MODEL), jnp.float32),
        in_specs=[pl.BlockSpec(memory_space=pltpu.VMEM)] * 5,
        out_specs=pl.BlockSpec(memory_space=pltpu.VMEM),
        scratch_shapes=[
            pltpu.VMEM((N_DEV * SQ, D_MODEL), jnp.float32),
            pltpu.VMEM((N_DEV * SQ, D_MODEL), jnp.float32),
            pltpu.VMEM((N_DEV * SQ, D_MODEL), jnp.float32),
            pltpu.VMEM((SQ, H * DH), jnp.float32),
            pltpu.VMEM((N_DEV * SQ, D_MODEL), jnp.float32),
            pltpu.VMEM(((N_DEV - 1) * SQ, D_MODEL), jnp.float32),
            pltpu.SemaphoreType.DMA((N_DEV - 1,)),
            pltpu.SemaphoreType.DMA((N_DEV - 1,)),
            pltpu.SemaphoreType.DMA((N_DEV - 1,)),
            pltpu.SemaphoreType.DMA((N_DEV - 1,)),
        ],
        compiler_params=pltpu.CompilerParams(collective_id=0),
    )(x2, Wq, Kp, Vp, Wo)

    return out.reshape(1, SQ, D_MODEL)


# baseline (device time: 164949 ns/iter reference)
import jax
import jax.numpy as jnp
from jax import lax
from jax.experimental import pallas as pl
from jax.experimental.pallas import tpu as pltpu

N_DEV = 4
SQ = 256
D_MODEL = 1024
H = 8
DH = 128
SKV = 4096
N_RES = 4
KV_PER_RES = SKV // N_RES
SCALE = 0.08838834764831843


def _body(x_ref, wq_ref, k_ref, v_ref, wo_ref, out_ref,
          xg_ref, xres_ref, qres_ref, ctx_ref, part_ref, rs_ref,
          ag_send, ag_recv, rs_send, rs_recv):
    my = lax.axis_index("i")
    right = lax.rem(my + 1, N_DEV)
    left = lax.rem(my + 3, N_DEV)

    barrier = pltpu.get_barrier_semaphore()
    pl.semaphore_signal(barrier, inc=1, device_id=(left,),
                        device_id_type=pl.DeviceIdType.MESH)
    pl.semaphore_signal(barrier, inc=1, device_id=(right,),
                        device_id_type=pl.DeviceIdType.MESH)
    pl.semaphore_wait(barrier, 2)

    xg_ref[pl.ds(my * SQ, SQ), :] = x_ref[:, :]
    for h in range(N_DEV - 1):
        c = lax.rem(my - h + N_DEV, N_DEV)
        rdma = pltpu.make_async_remote_copy(
            src_ref=xg_ref.at[pl.ds(c * SQ, SQ), :],
            dst_ref=xg_ref.at[pl.ds(c * SQ, SQ), :],
            send_sem=ag_send.at[h],
            recv_sem=ag_recv.at[h],
            device_id=(right,),
            device_id_type=pl.DeviceIdType.MESH,
        )
        rdma.start()
        rdma.wait()

    for j in range(N_DEV):
        for r in range(N_RES):
            xres_ref[r * SQ + j * 64:r * SQ + j * 64 + 64, :] = (
                xg_ref[j * SQ + r * 64:j * SQ + r * 64 + 64, :])

    qres_ref[:, :] = jnp.dot(xres_ref[:, :], wq_ref[:, :],
                             preferred_element_type=jnp.float32)

    for r in range(N_RES):
        q_r = qres_ref[r * SQ:(r + 1) * SQ, :]
        for h in range(H):
            qh = q_r[:, h * DH:(h + 1) * DH]
            k_rh = k_ref[r, h]
            s = lax.dot_general(
                qh, k_rh, (((1,), (1,)), ((), ())),
                preferred_element_type=jnp.float32) * SCALE
            m = jnp.max(s, axis=1, keepdims=True)
            e = jnp.exp(s - m)
            p = e / jnp.sum(e, axis=1, keepdims=True)
            ctx_ref[:, h * DH:(h + 1) * DH] = jnp.dot(
                p, v_ref[r, h], preferred_element_type=jnp.float32)
        acc = jnp.dot(ctx_ref[:, :], wo_ref[:, :],
                      preferred_element_type=jnp.float32)
        for j in range(N_DEV):
            part_ref[j * SQ + r * 64:j * SQ + r * 64 + 64, :] = (
                acc[j * 64:(j + 1) * 64, :])

    for s in range(N_DEV - 1):
        cs = lax.rem(my + 3 - s + N_DEV, N_DEV)
        cr = lax.rem(my + 2 - s + N_DEV, N_DEV)
        rdma = pltpu.make_async_remote_copy(
            src_ref=part_ref.at[pl.ds(cs * SQ, SQ), :],
            dst_ref=rs_ref.at[pl.ds(s * SQ, SQ), :],
            send_sem=rs_send.at[s],
            recv_sem=rs_recv.at[s],
            device_id=(right,),
            device_id_type=pl.DeviceIdType.MESH,
        )
        rdma.start()
        rdma.wait()
        part_ref[pl.ds(cr * SQ, SQ), :] = (
            part_ref[pl.ds(cr * SQ, SQ), :] + rs_ref[s * SQ:(s + 1) * SQ, :])

    out_ref[:, :] = part_ref[pl.ds(my * SQ, SQ), :]


def kernel(x, Wq, K_ext, V_ext, Wo):
    my = lax.axis_index("i")

    x2 = x[0]
    K = K_ext[0]
    V = V_ext[0]
    Kh = lax.dynamic_slice_in_dim(K, my * H, H, axis=1)
    Vh = lax.dynamic_slice_in_dim(V, my * H, H, axis=1)
    Kp = Kh.reshape(16, N_RES, 64, H, DH).transpose(1, 3, 0, 2, 4).reshape(
        N_RES, H, KV_PER_RES, DH)
    Vp = Vh.reshape(16, N_RES, 64, H, DH).transpose(1, 3, 0, 2, 4).reshape(
        N_RES, H, KV_PER_RES, DH)

    out = pl.pallas_call(
        _body,
        out_shape=jax.ShapeDtypeStruct((SQ, D_MODEL), jnp.float32),
        in_specs=[pl.BlockSpec(memory_space=pltpu.VMEM)] * 5,
        out_specs=pl.BlockSpec(memory_space=pltpu.VMEM),
        scratch_shapes=[
            pltpu.VMEM((N_DEV * SQ, D_MODEL), jnp.float32),
            pltpu.VMEM((N_DEV * SQ, D_MODEL), jnp.float32),
            pltpu.VMEM((N_DEV * SQ, D_MODEL), jnp.float32),
            pltpu.VMEM((SQ, H * DH), jnp.float32),
            pltpu.VMEM((N_DEV * SQ, D_MODEL), jnp.float32),
            pltpu.VMEM(((N_DEV - 1) * SQ, D_MODEL), jnp.float32),
            pltpu.SemaphoreType.DMA((N_DEV - 1,)),
            pltpu.SemaphoreType.DMA((N_DEV - 1,)),
            pltpu.SemaphoreType.DMA((N_DEV - 1,)),
            pltpu.SemaphoreType.DMA((N_DEV - 1,)),
        ],
        compiler_params=pltpu.CompilerParams(
            collective_id=0, vmem_limit_bytes=100 * 1024 * 1024),
    )(x2, Wq, Kp, Vp, Wo)

    return out.reshape(1, SQ, D_MODEL)


# device time: 153696 ns/iter; 1.0732x vs baseline; 1.0732x over previous
import jax
import jax.numpy as jnp
from jax import lax
from jax.experimental import pallas as pl
from jax.experimental.pallas import tpu as pltpu

N_DEV = 4
SQ = 256
D_MODEL = 1024
H = 8
DH = 128
SKV = 4096
N_RES = 4
KV_PER_RES = SKV // N_RES
SCALE = 0.08838834764831843


def _body(x_ref, wq_ref, k_ref, v_ref, wo_ref, out_ref,
          xg_ref, q_ref, ctx_ref, part_ref, rs_ref,
          ag_send, ag_recv, rs_send, rs_recv):
    my = lax.axis_index("i")
    right = lax.rem(my + 1, N_DEV)
    left = lax.rem(my + 3, N_DEV)

    barrier = pltpu.get_barrier_semaphore()
    pl.semaphore_signal(barrier, inc=1, device_id=(left,),
                        device_id_type=pl.DeviceIdType.MESH)
    pl.semaphore_signal(barrier, inc=1, device_id=(right,),
                        device_id_type=pl.DeviceIdType.MESH)
    pl.semaphore_wait(barrier, 2)

    def ag_rdma(h):
        c = lax.rem(my - h + N_DEV, N_DEV)
        return pltpu.make_async_remote_copy(
            src_ref=xg_ref.at[pl.ds(c * SQ, SQ), :],
            dst_ref=xg_ref.at[pl.ds(c * SQ, SQ), :],
            send_sem=ag_send.at[h],
            recv_sem=ag_recv.at[h],
            device_id=(right,),
            device_id_type=pl.DeviceIdType.MESH,
        )

    def rs_rdma(s):
        cs = lax.rem(my + 3 - s + N_DEV, N_DEV)
        return pltpu.make_async_remote_copy(
            src_ref=part_ref.at[pl.ds(cs * SQ, SQ), :],
            dst_ref=rs_ref.at[pl.ds(s * SQ, SQ), :],
            send_sem=rs_send.at[s],
            recv_sem=rs_recv.at[s],
            device_id=(right,),
            device_id_type=pl.DeviceIdType.MESH,
        )

    def rs_accum(s):
        cr = lax.rem(my + 2 - s + N_DEV, N_DEV)
        part_ref[pl.ds(cr * SQ, SQ), :] = (
            part_ref[pl.ds(cr * SQ, SQ), :] + rs_ref[s * SQ:(s + 1) * SQ, :])

    def compute_part(c):
        base = c * SQ
        q_ref[:, :] = jnp.dot(xg_ref[pl.ds(base, SQ), :], wq_ref[:, :],
                              preferred_element_type=jnp.float32)
        for r in range(N_RES):
            for h in range(H):
                qh = q_ref[r * 64:(r + 1) * 64, h * DH:(h + 1) * DH]
                s = lax.dot_general(
                    qh, k_ref[r, h], (((1,), (1,)), ((), ())),
                    preferred_element_type=jnp.float32) * SCALE
                m = jnp.max(s, axis=1, keepdims=True)
                e = jnp.exp(s - m)
                p = e / jnp.sum(e, axis=1, keepdims=True)
                ctx_ref[r * 64:(r + 1) * 64, h * DH:(h + 1) * DH] = jnp.dot(
                    p, v_ref[r, h], preferred_element_type=jnp.float32)
        part_ref[pl.ds(base, SQ), :] = jnp.dot(
            ctx_ref[:, :], wo_ref[:, :], preferred_element_type=jnp.float32)

    xg_ref[pl.ds(my * SQ, SQ), :] = x_ref[:, :]
    ag0 = ag_rdma(0)
    ag0.start()
    compute_part(my)
    ag0.wait_recv()
    ag1 = ag_rdma(1)
    ag1.start()
    compute_part(lax.rem(my + 3, N_DEV))
    rs0 = rs_rdma(0)
    rs0.start()
    ag1.wait_recv()
    ag2 = ag_rdma(2)
    ag2.start()
    compute_part(lax.rem(my + 2, N_DEV))
    rs0.wait_recv()
    rs_accum(0)
    rs1 = rs_rdma(1)
    rs1.start()
    ag2.wait_recv()
    compute_part(lax.rem(my + 1, N_DEV))
    rs1.wait_recv()
    rs_accum(1)
    rs2 = rs_rdma(2)
    rs2.start()
    rs2.wait_recv()
    rs_accum(2)

    out_ref[:, :] = part_ref[pl.ds(my * SQ, SQ), :]

    for d in (ag0, ag1, ag2, rs0, rs1, rs2):
        d.wait_send()


def kernel(x, Wq, K_ext, V_ext, Wo):
    my = lax.axis_index("i")

    x2 = x[0]
    K = K_ext[0]
    V = V_ext[0]
    Kh = lax.dynamic_slice_in_dim(K, my * H, H, axis=1)
    Vh = lax.dynamic_slice_in_dim(V, my * H, H, axis=1)
    Kp = Kh.reshape(16, N_RES, 64, H, DH).transpose(1, 3, 0, 2, 4).reshape(
        N_RES, H, KV_PER_RES, DH)
    Vp = Vh.reshape(16, N_RES, 64, H, DH).transpose(1, 3, 0, 2, 4).reshape(
        N_RES, H, KV_PER_RES, DH)

    out = pl.pallas_call(
        _body,
        out_shape=jax.ShapeDtypeStruct((SQ, D_MODEL), jnp.float32),
        in_specs=[pl.BlockSpec(memory_space=pltpu.VMEM)] * 5,
        out_specs=pl.BlockSpec(memory_space=pltpu.VMEM),
        scratch_shapes=[
            pltpu.VMEM((N_DEV * SQ, D_MODEL), jnp.float32),
            pltpu.VMEM((SQ, D_MODEL), jnp.float32),
            pltpu.VMEM((SQ, H * DH), jnp.float32),
            pltpu.VMEM((N_DEV * SQ, D_MODEL), jnp.float32),
            pltpu.VMEM(((N_DEV - 1) * SQ, D_MODEL), jnp.float32),
            pltpu.SemaphoreType.DMA((N_DEV - 1,)),
            pltpu.SemaphoreType.DMA((N_DEV - 1,)),
            pltpu.SemaphoreType.DMA((N_DEV - 1,)),
            pltpu.SemaphoreType.DMA((N_DEV - 1,)),
        ],
        compiler_params=pltpu.CompilerParams(
            collective_id=0, vmem_limit_bytes=100 * 1024 * 1024),
    )(x2, Wq, Kp, Vp, Wo)

    return out.reshape(1, SQ, D_MODEL)
